# Initial kernel scaffold; baseline (speedup 1.0000x reference)
import jax
import jax.numpy as jnp
from jax.experimental import pallas as pl


def kernel(x, edge_index, Wl1, bl1, Wr1, Wl2, bl2, Wr2, HW1, Hb1, HW2, Hb2, HW3, Hb3):
    raise NotImplementedError("write your pallas kernel here")


# speedup vs baseline: 9.5994x; 9.5994x over previous
import functools
import math

import jax
import jax.numpy as jnp
from jax import lax
from jax.experimental import pallas as pl
from jax.experimental.pallas import tpu as pltpu
from jax.experimental.pallas import tpu_sc as plsc

N_NODES = 10000
N_EDGES = 320000
F = 128
N_PAD = 10240


def _sc_pass1(x, src, dst, zf, zc):
    mesh = plsc.VectorSubcoreMesh(core_axis_name="c", subcore_axis_name="s")
    nc, ns = mesh.num_cores, mesh.num_subcores
    nw = nc * ns
    epw = N_EDGES // nw
    K = 80
    steps = epw // K
    rpt = (N_NODES // ns) // 8 * 8
    rem = N_NODES - rpt * ns
    cpt = N_PAD // ns

    @functools.partial(
        pl.kernel,
        out_type=(
            jax.ShapeDtypeStruct((nc, N_NODES, F), jnp.float32),
            jax.ShapeDtypeStruct((nc, N_PAD), jnp.float32),
        ),
        mesh=mesh,
        scratch_types=[
            pltpu.VMEM_SHARED((N_NODES, F), jnp.float32),
            pltpu.VMEM_SHARED((ns, N_PAD), jnp.float32),
            pltpu.VMEM((K,), jnp.int32),
            pltpu.VMEM((K,), jnp.int32),
            pltpu.VMEM((K, F), jnp.float32),
            pltpu.VMEM((N_PAD,), jnp.float32),
            pltpu.VMEM((ns, cpt), jnp.float32),
            pltpu.VMEM((cpt,), jnp.float32),
            pltpu.SemaphoreType.DMA,
        ],
        compiler_params=pltpu.CompilerParams(needs_layout_passes=False),
    )
    def k(x_hbm, src_hbm, dst_hbm, zf_hbm, zc_hbm, agg_out, cnt_out,
          agg_sh, cnt_stage, src_v, dst_v, rows_v, cnt_v, red_buf, red_out,
          sem):
        c = lax.axis_index("c")
        s = lax.axis_index("s")
        wid = c * ns + s

        pltpu.sync_copy(zf_hbm.at[pl.ds(s * rpt, rpt)],
                        agg_sh.at[pl.ds(s * rpt, rpt)])

        @pl.when(s == 0)
        def _():
            pltpu.sync_copy(zf_hbm.at[pl.ds(rpt * ns, rem)],
                            agg_sh.at[pl.ds(rpt * ns, rem)])

        pltpu.sync_copy(zc_hbm, cnt_v)
        plsc.subcore_barrier()

        ones16 = jnp.ones((16,), jnp.float32)

        def body(i, _):
            base = wid * epw + i * K
            pltpu.sync_copy(src_hbm.at[pl.ds(base, K)], src_v)
            pltpu.sync_copy(dst_hbm.at[pl.ds(base, K)], dst_v)
            pltpu.async_copy(x_hbm.at[src_v], rows_v, sem).wait()
            pltpu.sync_copy(rows_v, agg_sh.at[dst_v], add=True)
            for j in range(K // 16):
                idx = dst_v[pl.ds(j * 16, 16)]
                plsc.addupdate_scatter(cnt_v, [idx], ones16)
            return 0

        lax.fori_loop(0, steps, body, 0)

        pltpu.sync_copy(cnt_v, cnt_stage.at[s])
        plsc.subcore_barrier()

        pltpu.sync_copy(cnt_stage.at[:, pl.ds(s * cpt, cpt)], red_buf)

        def red(j, _):
            acc = red_buf[0, pl.ds(j * 16, 16)]
            for r in range(1, ns):
                acc = acc + red_buf[r, pl.ds(j * 16, 16)]
            red_out[pl.ds(j * 16, 16)] = acc
            return 0

        lax.fori_loop(0, cpt // 16, red, 0)
        pltpu.sync_copy(red_out, cnt_out.at[c, pl.ds(s * cpt, cpt)])

        pltpu.sync_copy(agg_sh.at[pl.ds(s * rpt, rpt)],
                        agg_out.at[c, pl.ds(s * rpt, rpt)])

        @pl.when(s == 0)
        def _():
            pltpu.sync_copy(agg_sh.at[pl.ds(rpt * ns, rem)],
                            agg_out.at[c, pl.ds(rpt * ns, rem)])

    return k(x, src, dst, zf, zc)


def _sc_pass2(cnt_p, src, dst, zc):
    mesh = plsc.VectorSubcoreMesh(core_axis_name="c", subcore_axis_name="s")
    nc, ns = mesh.num_cores, mesh.num_subcores
    nw = nc * ns
    epw = N_EDGES // nw
    cpt = N_PAD // ns

    @functools.partial(
        pl.kernel,
        out_type=(
            jax.ShapeDtypeStruct((nc, N_PAD), jnp.float32),
            jax.ShapeDtypeStruct((N_PAD,), jnp.float32),
        ),
        mesh=mesh,
        scratch_types=[
            pltpu.VMEM_SHARED((ns, N_PAD), jnp.float32),
            pltpu.VMEM((nc, N_PAD), jnp.float32),
            pltpu.VMEM((N_PAD,), jnp.float32),
            pltpu.VMEM((N_PAD,), jnp.float32),
            pltpu.VMEM((epw,), jnp.int32),
            pltpu.VMEM((epw,), jnp.int32),
            pltpu.VMEM((ns, cpt), jnp.float32),
            pltpu.VMEM((cpt,), jnp.float32),
        ],
        compiler_params=pltpu.CompilerParams(needs_layout_passes=False),
    )
    def k(cnt_hbm, src_hbm, dst_hbm, zc_hbm, w_out, inv_out,
          w_stage, cbuf, inv_v, w_v, src_v, dst_v, red_buf, red_out):
        c = lax.axis_index("c")
        s = lax.axis_index("s")
        wid = c * ns + s

        pltpu.sync_copy(cnt_hbm, cbuf)
        pltpu.sync_copy(zc_hbm, w_v)
        pltpu.sync_copy(src_hbm.at[pl.ds(wid * epw, epw)], src_v)
        pltpu.sync_copy(dst_hbm.at[pl.ds(wid * epw, epw)], dst_v)

        def inv_body(j, _):
            sl = pl.ds(j * 16, 16)
            tot = cbuf[0, sl] + cbuf[1, sl]
            inv_v[sl] = 1.0 / jnp.maximum(tot, 1.0)
            return 0

        lax.fori_loop(0, N_PAD // 16, inv_body, 0)

        def w_body(j, _):
            sl = pl.ds(j * 16, 16)
            vals = plsc.load_gather(inv_v, [dst_v[sl]])
            plsc.addupdate_scatter(w_v, [src_v[sl]], vals)
            return 0

        lax.fori_loop(0, epw // 16, w_body, 0)

        pltpu.sync_copy(w_v, w_stage.at[s])
        plsc.subcore_barrier()

        pltpu.sync_copy(w_stage.at[:, pl.ds(s * cpt, cpt)], red_buf)

        def red(j, _):
            acc = red_buf[0, pl.ds(j * 16, 16)]
            for r in range(1, ns):
                acc = acc + red_buf[r, pl.ds(j * 16, 16)]
            red_out[pl.ds(j * 16, 16)] = acc
            return 0

        lax.fori_loop(0, cpt // 16, red, 0)
        pltpu.sync_copy(red_out, w_out.at[c, pl.ds(s * cpt, cpt)])

        @pl.when(c == 0)
        def _():
            pltpu.sync_copy(inv_v.at[pl.ds(s * cpt, cpt)],
                            inv_out.at[pl.ds(s * cpt, cpt)])

    return k(cnt_p, src, dst, zc)


def _tc_dense(x, agg_p, inv_col, w_col, Wl1, bl1, Wr1, Wl2, bl2, Wr2,
              HW1, Hb1, HW2, Hb2, HW3, Hb3):
    BR = 1000
    grid = N_NODES // BR

    def body(x_ref, agg_ref, inv_ref, w_ref,
             Wl1_ref, bl1_ref, Wr1_ref, Wl2_ref, bl2_ref, Wr2_ref,
             HW1_ref, Hb1_ref, HW2_ref, Hb2_ref, HW3_ref, Hb3_ref,
             out_ref, acc0, acc1):
        i = pl.program_id(0)

        @pl.when(i == 0)
        def _():
            acc0[...] = jnp.zeros_like(acc0)
            acc1[...] = jnp.zeros_like(acc1)

        mean = (agg_ref[0] + agg_ref[1]) * inv_ref[...]
        h = mean @ Wl1_ref[...] + bl1_ref[...] + x_ref[...] @ Wr1_ref[...]
        h = jnp.maximum(h, 0.0)
        w = w_ref[0] + w_ref[1]
        acc0[...] += jnp.sum(h.reshape(BR // 8, 8, F), axis=0)
        acc1[...] += jnp.sum((w * h).reshape(BR // 8, 8, F), axis=0)

        @pl.when(i == grid - 1)
        def _():
            s0 = jnp.sum(acc0[...], axis=0, keepdims=True) / N_NODES
            s1 = jnp.sum(acc1[...], axis=0, keepdims=True) / N_NODES
            g = s1 @ Wl2_ref[...] + bl2_ref[...] + s0 @ Wr2_ref[...]
            t = jnp.maximum(g @ HW1_ref[...] + Hb1_ref[...], 0.0)
            t = jnp.maximum(t @ HW2_ref[...] + Hb2_ref[...], 0.0)
            t = t @ HW3_ref[...] + Hb3_ref[...]
            pi = jnp.float32(math.pi)
            out_ref[...] = jnp.mod(t + pi, 2.0 * pi) - pi

    full = lambda shape: pl.BlockSpec(shape, lambda i: (0,) * len(shape))
    return pl.pallas_call(
        body,
        grid=(grid,),
        in_specs=[
            pl.BlockSpec((BR, F), lambda i: (i, 0)),
            pl.BlockSpec((2, BR, F), lambda i: (0, i, 0)),
            pl.BlockSpec((BR, 1), lambda i: (i, 0)),
            pl.BlockSpec((2, BR, 1), lambda i: (0, i, 0)),
            full((F, F)), full((1, F)), full((F, F)),
            full((F, F)), full((1, F)), full((F, F)),
            full((F, 2 * F)), full((1, 2 * F)),
            full((2 * F, 2 * F)), full((1, 2 * F)),
            full((2 * F, 8)), full((1, 8)),
        ],
        out_specs=pl.BlockSpec((1, 8), lambda i: (0, 0)),
        out_shape=jax.ShapeDtypeStruct((1, 8), jnp.float32),
        scratch_shapes=[
            pltpu.VMEM((8, F), jnp.float32),
            pltpu.VMEM((8, F), jnp.float32),
        ],
    )(x, agg_p, inv_col, w_col, Wl1, bl1.reshape(1, F), Wr1,
      Wl2, bl2.reshape(1, F), Wr2,
      HW1, Hb1.reshape(1, 2 * F), HW2, Hb2.reshape(1, 2 * F),
      HW3, Hb3.reshape(1, 8))


def kernel(x, edge_index, Wl1, bl1, Wr1, Wl2, bl2, Wr2,
           HW1, Hb1, HW2, Hb2, HW3, Hb3):
    src = edge_index[0].astype(jnp.int32)
    dst = edge_index[1].astype(jnp.int32)
    zf = jnp.zeros((N_NODES, F), jnp.float32)
    zc = jnp.zeros((N_PAD,), jnp.float32)

    agg_p, cnt_p = _sc_pass1(x, src, dst, zf, zc)
    w_p, inv = _sc_pass2(cnt_p, src, dst, zc)
    t = _tc_dense(x, agg_p, inv.reshape(N_PAD, 1), w_p.reshape(2, N_PAD, 1),
                  Wl1, bl1, Wr1, Wl2, bl2, Wr2,
                  HW1, Hb1, HW2, Hb2, HW3, Hb3)
    return t[0]


# speedup vs baseline: 18.4334x; 1.9203x over previous
import functools
import math

import jax
import jax.numpy as jnp
from jax import lax
from jax.experimental import pallas as pl
from jax.experimental.pallas import tpu as pltpu
from jax.experimental.pallas import tpu_sc as plsc

N_NODES = 10000
N_EDGES = 320000
F = 128
N_PAD = 10240


def _sc_pass1(x, src, dst):
    mesh = plsc.VectorSubcoreMesh(core_axis_name="c", subcore_axis_name="s")
    nc, ns = mesh.num_cores, mesh.num_subcores
    nw = nc * ns
    epw = N_EDGES // nw
    K = 80
    steps = epw // K
    assert (steps - 1) % 4 == 0
    rpt = (N_NODES // ns) // 8 * 8
    rem = N_NODES - rpt * ns

    dst3 = dst.reshape(N_EDGES // K, 1, K)
    RB = 128
    SW = RB * ns

    @functools.partial(
        pl.kernel,
        out_type=(
            jax.ShapeDtypeStruct((nc, N_NODES, F), jnp.float32),
            jax.ShapeDtypeStruct((nc, N_PAD), jnp.float32),
        ),
        mesh=mesh,
        scratch_types=[
            pltpu.VMEM_SHARED((N_NODES, F), jnp.float32),
            pltpu.VMEM_SHARED((ns, SW), jnp.float32),
            [pltpu.VMEM((K,), jnp.int32) for _ in range(4)],
            [pltpu.VMEM((1, K), jnp.int32) for _ in range(4)],
            pltpu.VMEM((K, F), jnp.float32),
            pltpu.VMEM((K, F), jnp.float32),
            pltpu.VMEM((N_PAD,), jnp.float32),
            pltpu.VMEM((ns, RB), jnp.float32),
            pltpu.VMEM((RB,), jnp.float32),
            [pltpu.SemaphoreType.DMA for _ in range(4)],
            pltpu.SemaphoreType.DMA,
            pltpu.SemaphoreType.DMA,
        ],
        compiler_params=pltpu.CompilerParams(needs_layout_passes=False),
    )
    def k(x_hbm, src_hbm, dst_hbm, agg_out, cnt_out,
          agg_sh, cnt_stage, src_b, dst_b, rows0, rows1, cnt_v,
          red_buf, red_out, isem, g0, g1):
        c = lax.axis_index("c")
        s = lax.axis_index("s")
        wid = c * ns + s

        zero16 = jnp.zeros((16,), jnp.float32)
        ones16 = jnp.ones((16,), jnp.float32)

        def z0(t, _):
            rows0[t // 8, pl.ds((t % 8) * 16, 16)] = zero16
            return 0

        lax.fori_loop(0, K * F // 16, z0, 0)

        def zc(t, _):
            cnt_v[pl.ds(t * 16, 16)] = zero16
            return 0

        lax.fori_loop(0, N_PAD // 16, zc, 0)

        nfull = rpt // K
        for q in range(nfull):
            pltpu.sync_copy(rows0, agg_sh.at[pl.ds(s * rpt + q * K, K)])
        tailrows = rpt - nfull * K
        pltpu.sync_copy(rows0.at[pl.ds(0, tailrows)],
                        agg_sh.at[pl.ds(s * rpt + nfull * K, tailrows)])

        @pl.when(s == 0)
        def _():
            pltpu.sync_copy(rows0.at[pl.ds(0, rem)],
                            agg_sh.at[pl.ds(rpt * ns, rem)])

        plsc.subcore_barrier()

        rows = (rows0, rows1)
        gsem = (g0, g1)
        base_blk = wid * steps

        def idx_load(i, pos):
            gb = base_blk + i
            pltpu.async_copy(src_hbm.at[pl.ds(gb * K, K)], src_b[pos],
                             isem[pos])
            pltpu.async_copy(dst_hbm.at[gb], dst_b[pos], isem[pos])

        def idx_wait(pos):
            pltpu.make_async_copy(src_hbm.at[pl.ds(0, K)], src_b[pos],
                                  isem[pos]).wait()
            pltpu.make_async_copy(dst_hbm.at[0], dst_b[pos],
                                  isem[pos]).wait()

        def gather_issue(pos, rp):
            pltpu.async_copy(x_hbm.at[src_b[pos]], rows[rp], gsem[rp])

        def gather_wait(rp):
            pltpu.make_async_copy(x_hbm.at[src_b[0]], rows[rp],
                                  gsem[rp]).wait()

        def scatter_cnt(pos, rp):
            pltpu.sync_copy(rows[rp], agg_sh.at[dst_b[pos].at[0]], add=True)
            for j in range(K // 16):
                idx = dst_b[pos][0, pl.ds(j * 16, 16)]
                plsc.addupdate_scatter(cnt_v, [idx], ones16)

        for pos in range(4):
            idx_load(pos, pos)
        idx_wait(0)
        gather_issue(0, 0)

        def body(p, _):
            i0 = 4 * p
            for pos in range(4):
                i = i0 + pos
                nxt = (pos + 1) % 4
                idx_wait(nxt)
                gather_issue(nxt, (pos + 1) % 2)
                gather_wait(pos % 2)
                scatter_cnt(pos, pos % 2)
                if pos == 0:
                    idx_load(i + 4, pos)
                else:
                    @pl.when(i + 4 <= steps - 1)
                    def _(i=i, pos=pos):
                        idx_load(i + 4, pos)
            return 0

        lax.fori_loop(0, (steps - 1) // 4, body, 0)

        gather_wait(0)
        scatter_cnt(0, 0)

        plsc.subcore_barrier()
        for h in range(N_PAD // SW):
            pltpu.sync_copy(cnt_v.at[pl.ds(h * SW, SW)], cnt_stage.at[s])
            plsc.subcore_barrier()
            pltpu.sync_copy(cnt_stage.at[:, pl.ds(s * RB, RB)], red_buf)

            def red(j, _):
                acc = red_buf[0, pl.ds(j * 16, 16)]
                for r in range(1, ns):
                    acc = acc + red_buf[r, pl.ds(j * 16, 16)]
                red_out[pl.ds(j * 16, 16)] = acc
                return 0

            lax.fori_loop(0, RB // 16, red, 0)
            pltpu.sync_copy(red_out,
                            cnt_out.at[c, pl.ds(h * SW + s * RB, RB)])
            plsc.subcore_barrier()

        pltpu.sync_copy(agg_sh.at[pl.ds(s * rpt, rpt)],
                        agg_out.at[c, pl.ds(s * rpt, rpt)])

        @pl.when(s == 0)
        def _():
            pltpu.sync_copy(agg_sh.at[pl.ds(rpt * ns, rem)],
                            agg_out.at[c, pl.ds(rpt * ns, rem)])

    return k(x, src, dst3)


def _sc_pass2(cnt_p, src, dst):
    mesh = plsc.VectorSubcoreMesh(core_axis_name="c", subcore_axis_name="s")
    nc, ns = mesh.num_cores, mesh.num_subcores
    nw = nc * ns
    epw = N_EDGES // nw
    cpt = N_PAD // ns

    @functools.partial(
        pl.kernel,
        out_type=(
            jax.ShapeDtypeStruct((nc, N_PAD), jnp.float32),
            jax.ShapeDtypeStruct((N_PAD,), jnp.float32),
        ),
        mesh=mesh,
        scratch_types=[
            pltpu.VMEM_SHARED((ns, N_PAD), jnp.float32),
            pltpu.VMEM((nc, N_PAD), jnp.float32),
            pltpu.VMEM((N_PAD,), jnp.float32),
            pltpu.VMEM((N_PAD,), jnp.float32),
            pltpu.VMEM((epw,), jnp.int32),
            pltpu.VMEM((epw,), jnp.int32),
            pltpu.VMEM((ns, cpt), jnp.float32),
            pltpu.VMEM((cpt,), jnp.float32),
        ],
        compiler_params=pltpu.CompilerParams(needs_layout_passes=False),
    )
    def k(cnt_hbm, src_hbm, dst_hbm, w_out, inv_out,
          w_stage, cbuf, inv_v, w_v, src_v, dst_v, red_buf, red_out):
        c = lax.axis_index("c")
        s = lax.axis_index("s")
        wid = c * ns + s

        pltpu.sync_copy(cnt_hbm, cbuf)
        pltpu.sync_copy(src_hbm.at[pl.ds(wid * epw, epw)], src_v)
        pltpu.sync_copy(dst_hbm.at[pl.ds(wid * epw, epw)], dst_v)

        zero16 = jnp.zeros((16,), jnp.float32)

        def zw(t, _):
            w_v[pl.ds(t * 16, 16)] = zero16
            return 0

        lax.fori_loop(0, N_PAD // 16, zw, 0)

        def inv_body(j, _):
            sl = pl.ds(j * 16, 16)
            tot = cbuf[0, sl] + cbuf[1, sl]
            inv_v[sl] = 1.0 / jnp.maximum(tot, 1.0)
            return 0

        lax.fori_loop(0, N_PAD // 16, inv_body, 0)

        def w_body(j, _):
            sl = pl.ds(j * 16, 16)
            vals = plsc.load_gather(inv_v, [dst_v[sl]])
            plsc.addupdate_scatter(w_v, [src_v[sl]], vals)
            return 0

        lax.fori_loop(0, epw // 16, w_body, 0)

        pltpu.sync_copy(w_v, w_stage.at[s])
        plsc.subcore_barrier()

        pltpu.sync_copy(w_stage.at[:, pl.ds(s * cpt, cpt)], red_buf)

        def red(j, _):
            acc = red_buf[0, pl.ds(j * 16, 16)]
            for r in range(1, ns):
                acc = acc + red_buf[r, pl.ds(j * 16, 16)]
            red_out[pl.ds(j * 16, 16)] = acc
            return 0

        lax.fori_loop(0, cpt // 16, red, 0)
        pltpu.sync_copy(red_out, w_out.at[c, pl.ds(s * cpt, cpt)])

        @pl.when(c == 0)
        def _():
            pltpu.sync_copy(inv_v.at[pl.ds(s * cpt, cpt)],
                            inv_out.at[pl.ds(s * cpt, cpt)])

    return k(cnt_p, src, dst)


def _tc_dense(x, agg_p, inv_col, w_col, Wl1, bl1, Wr1, Wl2, bl2, Wr2,
              HW1, Hb1, HW2, Hb2, HW3, Hb3):
    BR = 1000
    grid = N_NODES // BR

    def body(x_ref, agg_ref, inv_ref, w_ref,
             Wl1_ref, bl1_ref, Wr1_ref, Wl2_ref, bl2_ref, Wr2_ref,
             HW1_ref, Hb1_ref, HW2_ref, Hb2_ref, HW3_ref, Hb3_ref,
             out_ref, acc0, acc1):
        i = pl.program_id(0)

        @pl.when(i == 0)
        def _():
            acc0[...] = jnp.zeros_like(acc0)
            acc1[...] = jnp.zeros_like(acc1)

        mean = (agg_ref[0] + agg_ref[1]) * inv_ref[...]
        h = mean @ Wl1_ref[...] + bl1_ref[...] + x_ref[...] @ Wr1_ref[...]
        h = jnp.maximum(h, 0.0)
        w = w_ref[0] + w_ref[1]
        acc0[...] += jnp.sum(h.reshape(BR // 8, 8, F), axis=0)
        acc1[...] += jnp.sum((w * h).reshape(BR // 8, 8, F), axis=0)

        @pl.when(i == grid - 1)
        def _():
            s0 = jnp.sum(acc0[...], axis=0, keepdims=True) / N_NODES
            s1 = jnp.sum(acc1[...], axis=0, keepdims=True) / N_NODES
            g = s1 @ Wl2_ref[...] + bl2_ref[...] + s0 @ Wr2_ref[...]
            t = jnp.maximum(g @ HW1_ref[...] + Hb1_ref[...], 0.0)
            t = jnp.maximum(t @ HW2_ref[...] + Hb2_ref[...], 0.0)
            t = t @ HW3_ref[...] + Hb3_ref[...]
            pi = jnp.float32(math.pi)
            out_ref[...] = jnp.mod(t + pi, 2.0 * pi) - pi

    full = lambda shape: pl.BlockSpec(shape, lambda i: (0,) * len(shape))
    return pl.pallas_call(
        body,
        grid=(grid,),
        in_specs=[
            pl.BlockSpec((BR, F), lambda i: (i, 0)),
            pl.BlockSpec((2, BR, F), lambda i: (0, i, 0)),
            pl.BlockSpec((BR, 1), lambda i: (i, 0)),
            pl.BlockSpec((2, BR, 1), lambda i: (0, i, 0)),
            full((F, F)), full((1, F)), full((F, F)),
            full((F, F)), full((1, F)), full((F, F)),
            full((F, 2 * F)), full((1, 2 * F)),
            full((2 * F, 2 * F)), full((1, 2 * F)),
            full((2 * F, 8)), full((1, 8)),
        ],
        out_specs=pl.BlockSpec((1, 8), lambda i: (0, 0)),
        out_shape=jax.ShapeDtypeStruct((1, 8), jnp.float32),
        scratch_shapes=[
            pltpu.VMEM((8, F), jnp.float32),
            pltpu.VMEM((8, F), jnp.float32),
        ],
    )(x, agg_p, inv_col, w_col, Wl1, bl1.reshape(1, F), Wr1,
      Wl2, bl2.reshape(1, F), Wr2,
      HW1, Hb1.reshape(1, 2 * F), HW2, Hb2.reshape(1, 2 * F),
      HW3, Hb3.reshape(1, 8))


def kernel(x, edge_index, Wl1, bl1, Wr1, Wl2, bl2, Wr2,
           HW1, Hb1, HW2, Hb2, HW3, Hb3):
    src = edge_index[0].astype(jnp.int32)
    dst = edge_index[1].astype(jnp.int32)

    agg_p, cnt_p = _sc_pass1(x, src, dst)
    w_p, inv = _sc_pass2(cnt_p, src, dst)
    t = _tc_dense(x, agg_p, inv.reshape(N_PAD, 1), w_p.reshape(2, N_PAD, 1),
                  Wl1, bl1, Wr1, Wl2, bl2, Wr2,
                  HW1, Hb1, HW2, Hb2, HW3, Hb3)
    return t[0]


# speedup vs baseline: 19.6957x; 1.0685x over previous
import functools
import math

import jax
import jax.numpy as jnp
from jax import lax
from jax.experimental import pallas as pl
from jax.experimental.pallas import tpu as pltpu
from jax.experimental.pallas import tpu_sc as plsc

N_NODES = 10000
N_EDGES = 320000
F = 128
N_PAD = 10240


def _sc_pass1(x, ei3):
    mesh = plsc.VectorSubcoreMesh(core_axis_name="c", subcore_axis_name="s")
    nc, ns = mesh.num_cores, mesh.num_subcores
    nw = nc * ns
    epw = N_EDGES // nw
    K = 80
    steps = epw // K
    assert (steps - 1) % 4 == 0
    rpt = (N_NODES // ns) // 8 * 8
    rem = N_NODES - rpt * ns
    nblk = N_EDGES // K

    RB = 128
    SW = RB * ns

    @functools.partial(
        pl.kernel,
        out_type=(
            jax.ShapeDtypeStruct((nc, N_NODES, F), jnp.float32),
            jax.ShapeDtypeStruct((nc, N_PAD), jnp.float32),
        ),
        mesh=mesh,
        scratch_types=[
            pltpu.VMEM_SHARED((N_NODES, F), jnp.float32),
            pltpu.VMEM_SHARED((ns, SW), jnp.float32),
            [pltpu.VMEM((1, K), jnp.int32) for _ in range(4)],
            [pltpu.VMEM((1, K), jnp.int32) for _ in range(4)],
            pltpu.VMEM((K, F), jnp.float32),
            pltpu.VMEM((K, F), jnp.float32),
            pltpu.VMEM((N_PAD,), jnp.float32),
            pltpu.VMEM((ns, RB), jnp.float32),
            pltpu.VMEM((RB,), jnp.float32),
            [pltpu.SemaphoreType.DMA for _ in range(4)],
            pltpu.SemaphoreType.DMA,
            pltpu.SemaphoreType.DMA,
            pltpu.SemaphoreType.DMA,
            pltpu.SemaphoreType.DMA,
        ],
        compiler_params=pltpu.CompilerParams(needs_layout_passes=False),
    )
    def k(x_hbm, ei_hbm, agg_out, cnt_out,
          agg_sh, cnt_stage, src_b, dst_b, rows0, rows1, cnt_v,
          red_buf, red_out, isem, g0, g1, ss0, ss1):
        c = lax.axis_index("c")
        s = lax.axis_index("s")
        wid = c * ns + s

        zero16 = jnp.zeros((16,), jnp.float32)
        ones16 = jnp.ones((16,), jnp.float32)

        def z0(t, _):
            rows0[t // 8, pl.ds((t % 8) * 16, 16)] = zero16
            return 0

        lax.fori_loop(0, K * F // 16, z0, 0)

        def zc(t, _):
            cnt_v[pl.ds(t * 16, 16)] = zero16
            return 0

        lax.fori_loop(0, N_PAD // 16, zc, 0)

        nfull = rpt // K
        for q in range(nfull):
            pltpu.sync_copy(rows0, agg_sh.at[pl.ds(s * rpt + q * K, K)])
        tailrows = rpt - nfull * K
        pltpu.sync_copy(rows0.at[pl.ds(0, tailrows)],
                        agg_sh.at[pl.ds(s * rpt + nfull * K, tailrows)])

        @pl.when(s == 0)
        def _():
            pltpu.sync_copy(rows0.at[pl.ds(0, rem)],
                            agg_sh.at[pl.ds(rpt * ns, rem)])

        plsc.subcore_barrier()

        rows = (rows0, rows1)
        gsem = (g0, g1)
        ssem = (ss0, ss1)
        base_blk = wid * steps

        def idx_load(i, pos):
            gb = base_blk + i
            pltpu.async_copy(ei_hbm.at[gb], src_b[pos], isem[pos])
            pltpu.async_copy(ei_hbm.at[nblk + gb], dst_b[pos], isem[pos])

        def idx_wait(pos):
            pltpu.make_async_copy(ei_hbm.at[0], src_b[pos], isem[pos]).wait()
            pltpu.make_async_copy(ei_hbm.at[0], dst_b[pos], isem[pos]).wait()

        def gather_issue(pos, rp):
            pltpu.async_copy(x_hbm.at[src_b[pos].at[0]], rows[rp], gsem[rp])

        def gather_wait(rp):
            pltpu.make_async_copy(x_hbm.at[src_b[0].at[0]], rows[rp],
                                  gsem[rp]).wait()

        def scatter_issue(pos, rp):
            pltpu.async_copy(rows[rp], agg_sh.at[dst_b[pos].at[0]], ssem[rp],
                             add=True)

        def scatter_wait(rp):
            pltpu.make_async_copy(rows[rp], agg_sh.at[dst_b[0].at[0]],
                                  ssem[rp]).wait()

        def cnt_upd(pos):
            for j in range(K // 16):
                idx = dst_b[pos][0, pl.ds(j * 16, 16)]
                plsc.addupdate_scatter(cnt_v, [idx], ones16)

        for pos in range(4):
            idx_load(pos, pos)
        idx_wait(0)
        gather_issue(0, 0)

        idx_wait(1)
        gather_issue(1, 1)
        gather_wait(0)
        scatter_issue(0, 0)
        cnt_upd(0)

        nbody = (steps - 1) // 4

        def body(p, _):
            for q in range(4):
                i = 4 * p + 1 + q
                pos = (q + 1) % 4
                nxt = (pos + 1) % 4
                rp = (q + 1) % 2
                nrp = q % 2

                @pl.when(i + 1 <= steps - 1)
                def _(pos=pos, nxt=nxt, nrp=nrp):
                    idx_wait(nxt)

                scatter_wait(nrp)

                @pl.when(i + 1 <= steps - 1)
                def _(pos=pos, nxt=nxt, nrp=nrp):
                    gather_issue(nxt, nrp)

                gather_wait(rp)
                scatter_issue(pos, rp)
                cnt_upd(pos)

                @pl.when(i + 3 <= steps - 1)
                def _(i=i, q=q):
                    idx_load(i + 3, q)

            return 0

        lax.fori_loop(0, nbody, body, 0)

        scatter_wait(0)

        plsc.subcore_barrier()
        for h in range(N_PAD // SW):
            pltpu.sync_copy(cnt_v.at[pl.ds(h * SW, SW)], cnt_stage.at[s])
            plsc.subcore_barrier()
            pltpu.sync_copy(cnt_stage.at[:, pl.ds(s * RB, RB)], red_buf)

            def red(j, _):
                acc = red_buf[0, pl.ds(j * 16, 16)]
                for r in range(1, ns):
                    acc = acc + red_buf[r, pl.ds(j * 16, 16)]
                red_out[pl.ds(j * 16, 16)] = acc
                return 0

            lax.fori_loop(0, RB // 16, red, 0)
            pltpu.sync_copy(red_out,
                            cnt_out.at[c, pl.ds(h * SW + s * RB, RB)])
            plsc.subcore_barrier()

        pltpu.sync_copy(agg_sh.at[pl.ds(s * rpt, rpt)],
                        agg_out.at[c, pl.ds(s * rpt, rpt)])

        @pl.when(s == 0)
        def _():
            pltpu.sync_copy(agg_sh.at[pl.ds(rpt * ns, rem)],
                            agg_out.at[c, pl.ds(rpt * ns, rem)])

    return k(x, ei3)


def _sc_pass2(cnt_p, ei):
    mesh = plsc.VectorSubcoreMesh(core_axis_name="c", subcore_axis_name="s")
    nc, ns = mesh.num_cores, mesh.num_subcores
    nw = nc * ns
    epw = N_EDGES // nw
    cpt = N_PAD // ns

    @functools.partial(
        pl.kernel,
        out_type=(
            jax.ShapeDtypeStruct((nc, N_PAD), jnp.float32),
            jax.ShapeDtypeStruct((N_PAD,), jnp.float32),
        ),
        mesh=mesh,
        scratch_types=[
            pltpu.VMEM_SHARED((ns, N_PAD), jnp.float32),
            pltpu.VMEM((nc, N_PAD), jnp.float32),
            pltpu.VMEM((N_PAD,), jnp.float32),
            pltpu.VMEM((N_PAD,), jnp.float32),
            pltpu.VMEM((epw,), jnp.int32),
            pltpu.VMEM((epw,), jnp.int32),
            pltpu.VMEM((ns, cpt), jnp.float32),
            pltpu.VMEM((cpt,), jnp.float32),
        ],
        compiler_params=pltpu.CompilerParams(needs_layout_passes=False),
    )
    def k(cnt_hbm, ei_hbm, w_out, inv_out,
          w_stage, cbuf, inv_v, w_v, src_v, dst_v, red_buf, red_out):
        c = lax.axis_index("c")
        s = lax.axis_index("s")
        wid = c * ns + s

        pltpu.sync_copy(cnt_hbm, cbuf)
        pltpu.sync_copy(ei_hbm.at[pl.ds(wid * epw, epw)], src_v)
        pltpu.sync_copy(ei_hbm.at[pl.ds(N_EDGES + wid * epw, epw)], dst_v)

        zero16 = jnp.zeros((16,), jnp.float32)

        def zw(t, _):
            w_v[pl.ds(t * 16, 16)] = zero16
            return 0

        lax.fori_loop(0, N_PAD // 16, zw, 0)

        def inv_body(j, _):
            sl = pl.ds(j * 16, 16)
            tot = cbuf[0, sl] + cbuf[1, sl]
            inv_v[sl] = 1.0 / jnp.maximum(tot, 1.0)
            return 0

        lax.fori_loop(0, N_PAD // 16, inv_body, 0)

        def w_body(j, _):
            sl = pl.ds(j * 16, 16)
            vals = plsc.load_gather(inv_v, [dst_v[sl]])
            plsc.addupdate_scatter(w_v, [src_v[sl]], vals)
            return 0

        lax.fori_loop(0, epw // 16, w_body, 0)

        pltpu.sync_copy(w_v, w_stage.at[s])
        plsc.subcore_barrier()

        pltpu.sync_copy(w_stage.at[:, pl.ds(s * cpt, cpt)], red_buf)

        def red(j, _):
            acc = red_buf[0, pl.ds(j * 16, 16)]
            for r in range(1, ns):
                acc = acc + red_buf[r, pl.ds(j * 16, 16)]
            red_out[pl.ds(j * 16, 16)] = acc
            return 0

        lax.fori_loop(0, cpt // 16, red, 0)
        pltpu.sync_copy(red_out, w_out.at[c, pl.ds(s * cpt, cpt)])

        @pl.when(c == 0)
        def _():
            pltpu.sync_copy(inv_v.at[pl.ds(s * cpt, cpt)],
                            inv_out.at[pl.ds(s * cpt, cpt)])

    return k(cnt_p, ei)


def _tc_dense(x, agg_p, inv_col, w_col, Wl1, bl1, Wr1, Wl2, bl2, Wr2,
              HW1, Hb1, HW2, Hb2, HW3, Hb3):
    BR = 1000
    grid = N_NODES // BR

    def body(x_ref, agg_ref, inv_ref, w_ref,
             Wl1_ref, bl1_ref, Wr1_ref, Wl2_ref, bl2_ref, Wr2_ref,
             HW1_ref, Hb1_ref, HW2_ref, Hb2_ref, HW3_ref, Hb3_ref,
             out_ref, acc0, acc1):
        i = pl.program_id(0)

        @pl.when(i == 0)
        def _():
            acc0[...] = jnp.zeros_like(acc0)
            acc1[...] = jnp.zeros_like(acc1)

        mean = (agg_ref[0] + agg_ref[1]) * inv_ref[...]
        h = mean @ Wl1_ref[...] + bl1_ref[...] + x_ref[...] @ Wr1_ref[...]
        h = jnp.maximum(h, 0.0)
        w = w_ref[0] + w_ref[1]
        acc0[...] += jnp.sum(h.reshape(BR // 8, 8, F), axis=0)
        acc1[...] += jnp.sum((w * h).reshape(BR // 8, 8, F), axis=0)

        @pl.when(i == grid - 1)
        def _():
            s0 = jnp.sum(acc0[...], axis=0, keepdims=True) / N_NODES
            s1 = jnp.sum(acc1[...], axis=0, keepdims=True) / N_NODES
            g = s1 @ Wl2_ref[...] + bl2_ref[...] + s0 @ Wr2_ref[...]
            t = jnp.maximum(g @ HW1_ref[...] + Hb1_ref[...], 0.0)
            t = jnp.maximum(t @ HW2_ref[...] + Hb2_ref[...], 0.0)
            t = t @ HW3_ref[...] + Hb3_ref[...]
            pi = jnp.float32(math.pi)
            out_ref[...] = jnp.mod(t + pi, 2.0 * pi) - pi

    full = lambda shape: pl.BlockSpec(shape, lambda i: (0,) * len(shape))
    return pl.pallas_call(
        body,
        grid=(grid,),
        in_specs=[
            pl.BlockSpec((BR, F), lambda i: (i, 0)),
            pl.BlockSpec((2, BR, F), lambda i: (0, i, 0)),
            pl.BlockSpec((BR, 1), lambda i: (i, 0)),
            pl.BlockSpec((2, BR, 1), lambda i: (0, i, 0)),
            full((F, F)), full((1, F)), full((F, F)),
            full((F, F)), full((1, F)), full((F, F)),
            full((F, 2 * F)), full((1, 2 * F)),
            full((2 * F, 2 * F)), full((1, 2 * F)),
            full((2 * F, 8)), full((1, 8)),
        ],
        out_specs=pl.BlockSpec((1, 8), lambda i: (0, 0)),
        out_shape=jax.ShapeDtypeStruct((1, 8), jnp.float32),
        scratch_shapes=[
            pltpu.VMEM((8, F), jnp.float32),
            pltpu.VMEM((8, F), jnp.float32),
        ],
    )(x, agg_p, inv_col, w_col, Wl1, bl1.reshape(1, F), Wr1,
      Wl2, bl2.reshape(1, F), Wr2,
      HW1, Hb1.reshape(1, 2 * F), HW2, Hb2.reshape(1, 2 * F),
      HW3, Hb3.reshape(1, 8))


def kernel(x, edge_index, Wl1, bl1, Wr1, Wl2, bl2, Wr2,
           HW1, Hb1, HW2, Hb2, HW3, Hb3):
    ei32 = edge_index.astype(jnp.int32)
    ei3 = ei32.reshape(2 * N_EDGES // 80, 1, 80)
    ei_flat = ei32.reshape(2 * N_EDGES)

    agg_p, cnt_p = _sc_pass1(x, ei3)
    w_p, inv = _sc_pass2(cnt_p, ei_flat)
    t = _tc_dense(x, agg_p, inv.reshape(N_PAD, 1), w_p.reshape(2, N_PAD, 1),
                  Wl1, bl1, Wr1, Wl2, bl2, Wr2,
                  HW1, Hb1, HW2, Hb2, HW3, Hb3)
    return t[0]


# speedup vs baseline: 19.8737x; 1.0090x over previous
import functools
import math

import jax
import jax.numpy as jnp
from jax import lax
from jax.experimental import pallas as pl
from jax.experimental.pallas import tpu as pltpu
from jax.experimental.pallas import tpu_sc as plsc

N_NODES = 10000
N_EDGES = 320000
F = 128
N_PAD = 10240


def _sc_pass1(x, ei3):
    mesh = plsc.VectorSubcoreMesh(core_axis_name="c", subcore_axis_name="s")
    nc, ns = mesh.num_cores, mesh.num_subcores
    nw = nc * ns
    epw = N_EDGES // nw
    K = 80
    steps = epw // K
    assert (steps - 1) % 4 == 0
    rpt = (N_NODES // ns) // 8 * 8
    rem = N_NODES - rpt * ns
    nblk = N_EDGES // K

    RB = 128
    SW = RB * ns

    @functools.partial(
        pl.kernel,
        out_type=(
            jax.ShapeDtypeStruct((nc, N_NODES, F), jnp.float32),
            jax.ShapeDtypeStruct((nc, N_PAD), jnp.float32),
        ),
        mesh=mesh,
        scratch_types=[
            pltpu.VMEM_SHARED((N_NODES, F), jnp.float32),
            pltpu.VMEM_SHARED((ns, SW), jnp.float32),
            [pltpu.VMEM((1, K), jnp.int32) for _ in range(4)],
            [pltpu.VMEM((1, K), jnp.int32) for _ in range(4)],
            pltpu.VMEM((K, F), jnp.float32),
            pltpu.VMEM((K, F), jnp.float32),
            pltpu.VMEM((N_PAD,), jnp.float32),
            pltpu.VMEM((ns, RB), jnp.float32),
            pltpu.VMEM((RB,), jnp.float32),
            [pltpu.SemaphoreType.DMA for _ in range(4)],
            pltpu.SemaphoreType.DMA,
            pltpu.SemaphoreType.DMA,
            pltpu.SemaphoreType.DMA,
            pltpu.SemaphoreType.DMA,
        ],
        compiler_params=pltpu.CompilerParams(needs_layout_passes=False),
    )
    def k(x_hbm, ei_hbm, agg_out, cnt_out,
          agg_sh, cnt_stage, src_b, dst_b, rows0, rows1, cnt_v,
          red_buf, red_out, isem, g0, g1, ss0, ss1):
        c = lax.axis_index("c")
        s = lax.axis_index("s")
        wid = c * ns + s

        zero16 = jnp.zeros((16,), jnp.float32)
        ones16 = jnp.ones((16,), jnp.float32)

        def z0(t, _):
            rows0[t // 8, pl.ds((t % 8) * 16, 16)] = zero16
            return 0

        lax.fori_loop(0, K * F // 16, z0, 0)

        def zc(t, _):
            cnt_v[pl.ds(t * 16, 16)] = zero16
            return 0

        lax.fori_loop(0, N_PAD // 16, zc, 0)

        nfull = rpt // K
        for q in range(nfull):
            pltpu.sync_copy(rows0, agg_sh.at[pl.ds(s * rpt + q * K, K)])
        tailrows = rpt - nfull * K
        pltpu.sync_copy(rows0.at[pl.ds(0, tailrows)],
                        agg_sh.at[pl.ds(s * rpt + nfull * K, tailrows)])

        @pl.when(s == 0)
        def _():
            pltpu.sync_copy(rows0.at[pl.ds(0, rem)],
                            agg_sh.at[pl.ds(rpt * ns, rem)])

        plsc.subcore_barrier()

        rows = (rows0, rows1)
        gsem = (g0, g1)
        ssem = (ss0, ss1)
        base_blk = wid * steps

        def idx_load(i, pos):
            gb = base_blk + i
            pltpu.async_copy(ei_hbm.at[gb], src_b[pos], isem[pos])
            pltpu.async_copy(ei_hbm.at[nblk + gb], dst_b[pos], isem[pos])

        def idx_wait(pos):
            pltpu.make_async_copy(ei_hbm.at[0], src_b[pos], isem[pos]).wait()
            pltpu.make_async_copy(ei_hbm.at[0], dst_b[pos], isem[pos]).wait()

        def gather_issue(pos, rp):
            pltpu.async_copy(x_hbm.at[src_b[pos].at[0]], rows[rp], gsem[rp])

        def gather_wait(rp):
            pltpu.make_async_copy(x_hbm.at[src_b[0].at[0]], rows[rp],
                                  gsem[rp]).wait()

        def scatter_issue(pos, rp):
            pltpu.async_copy(rows[rp], agg_sh.at[dst_b[pos].at[0]], ssem[rp],
                             add=True)

        def scatter_wait(rp):
            pltpu.make_async_copy(rows[rp], agg_sh.at[dst_b[0].at[0]],
                                  ssem[rp]).wait()

        def cnt_upd(pos):
            for j in range(K // 16):
                idx = dst_b[pos][0, pl.ds(j * 16, 16)]
                plsc.addupdate_scatter(cnt_v, [idx], ones16)

        for pos in range(4):
            idx_load(pos, pos)
        idx_wait(0)
        gather_issue(0, 0)

        idx_wait(1)
        gather_issue(1, 1)
        gather_wait(0)
        scatter_issue(0, 0)
        cnt_upd(0)

        nbody = (steps - 1) // 4

        def body(p, _):
            for q in range(4):
                i = 4 * p + 1 + q
                pos = (q + 1) % 4
                nxt = (pos + 1) % 4
                rp = (q + 1) % 2
                nrp = q % 2

                @pl.when(i + 1 <= steps - 1)
                def _(pos=pos, nxt=nxt, nrp=nrp):
                    idx_wait(nxt)

                scatter_wait(nrp)

                @pl.when(i + 1 <= steps - 1)
                def _(pos=pos, nxt=nxt, nrp=nrp):
                    gather_issue(nxt, nrp)

                gather_wait(rp)
                scatter_issue(pos, rp)
                cnt_upd(pos)

                @pl.when(i + 3 <= steps - 1)
                def _(i=i, q=q):
                    idx_load(i + 3, q)

            return 0

        lax.fori_loop(0, nbody, body, 0)

        scatter_wait(0)

        plsc.subcore_barrier()
        for h in range(N_PAD // SW):
            pltpu.sync_copy(cnt_v.at[pl.ds(h * SW, SW)], cnt_stage.at[s])
            plsc.subcore_barrier()
            pltpu.sync_copy(cnt_stage.at[:, pl.ds(s * RB, RB)], red_buf)

            def red(j, _):
                acc = red_buf[0, pl.ds(j * 16, 16)]
                for r in range(1, ns):
                    acc = acc + red_buf[r, pl.ds(j * 16, 16)]
                red_out[pl.ds(j * 16, 16)] = acc
                return 0

            lax.fori_loop(0, RB // 16, red, 0)
            pltpu.sync_copy(red_out,
                            cnt_out.at[c, pl.ds(h * SW + s * RB, RB)])
            plsc.subcore_barrier()

        pltpu.sync_copy(agg_sh.at[pl.ds(s * rpt, rpt)],
                        agg_out.at[c, pl.ds(s * rpt, rpt)])

        @pl.when(s == 0)
        def _():
            pltpu.sync_copy(agg_sh.at[pl.ds(rpt * ns, rem)],
                            agg_out.at[c, pl.ds(rpt * ns, rem)])

    return k(x, ei3)


def _sc_pass2(cnt_p, ei):
    mesh = plsc.VectorSubcoreMesh(core_axis_name="c", subcore_axis_name="s")
    nc, ns = mesh.num_cores, mesh.num_subcores
    nw = nc * ns
    epw = N_EDGES // nw
    cpt = N_PAD // ns

    @functools.partial(
        pl.kernel,
        out_type=(
            jax.ShapeDtypeStruct((nc, N_PAD), jnp.float32),
            jax.ShapeDtypeStruct((N_PAD,), jnp.float32),
        ),
        mesh=mesh,
        scratch_types=[
            pltpu.VMEM_SHARED((ns, N_PAD), jnp.float32),
            pltpu.VMEM((nc, N_PAD), jnp.float32),
            pltpu.VMEM((N_PAD,), jnp.float32),
            pltpu.VMEM((N_PAD,), jnp.float32),
            pltpu.VMEM((epw,), jnp.int32),
            pltpu.VMEM((epw,), jnp.int32),
            pltpu.VMEM((ns, cpt), jnp.float32),
            pltpu.VMEM((cpt,), jnp.float32),
            pltpu.SemaphoreType.DMA,
            pltpu.SemaphoreType.DMA,
        ],
        compiler_params=pltpu.CompilerParams(needs_layout_passes=False),
    )
    def k(cnt_hbm, ei_hbm, w_out, inv_out,
          w_stage, cbuf, inv_v, w_v, src_v, dst_v, red_buf, red_out,
          csem, esem):
        c = lax.axis_index("c")
        s = lax.axis_index("s")
        wid = c * ns + s

        pltpu.async_copy(cnt_hbm, cbuf, csem)
        pltpu.async_copy(ei_hbm.at[pl.ds(wid * epw, epw)], src_v, esem)
        pltpu.async_copy(ei_hbm.at[pl.ds(N_EDGES + wid * epw, epw)], dst_v,
                         esem)

        zero16 = jnp.zeros((16,), jnp.float32)

        def zw(t, _):
            for u in range(5):
                w_v[pl.ds((t * 5 + u) * 16, 16)] = zero16
            return 0

        lax.fori_loop(0, N_PAD // 80, zw, 0)

        pltpu.make_async_copy(cnt_hbm, cbuf, csem).wait()

        def inv_body(j, _):
            for u in range(5):
                sl = pl.ds((j * 5 + u) * 16, 16)
                tot = cbuf[0, sl] + cbuf[1, sl]
                inv_v[sl] = 1.0 / jnp.maximum(tot, 1.0)
            return 0

        lax.fori_loop(0, N_PAD // 80, inv_body, 0)

        pltpu.make_async_copy(ei_hbm.at[pl.ds(0, epw)], src_v, esem).wait()
        pltpu.make_async_copy(ei_hbm.at[pl.ds(0, epw)], dst_v, esem).wait()

        def w_body(j, _):
            for u in range(5):
                sl = pl.ds((j * 5 + u) * 16, 16)
                vals = plsc.load_gather(inv_v, [dst_v[sl]])
                plsc.addupdate_scatter(w_v, [src_v[sl]], vals)
            return 0

        lax.fori_loop(0, epw // 80, w_body, 0)

        pltpu.sync_copy(w_v, w_stage.at[s])
        plsc.subcore_barrier()

        pltpu.sync_copy(w_stage.at[:, pl.ds(s * cpt, cpt)], red_buf)

        def red(j, _):
            acc = red_buf[0, pl.ds(j * 16, 16)]
            for r in range(1, ns):
                acc = acc + red_buf[r, pl.ds(j * 16, 16)]
            red_out[pl.ds(j * 16, 16)] = acc
            return 0

        lax.fori_loop(0, cpt // 16, red, 0)
        pltpu.sync_copy(red_out, w_out.at[c, pl.ds(s * cpt, cpt)])

        @pl.when(c == 0)
        def _():
            pltpu.sync_copy(inv_v.at[pl.ds(s * cpt, cpt)],
                            inv_out.at[pl.ds(s * cpt, cpt)])

    return k(cnt_p, ei)


def _tc_dense(x, agg_p, Wl1, bl1, Wr1):
    BR = 1000
    grid = N_NODES // BR

    def body(x_ref, agg_ref, Wl1_ref, bl1_ref, Wr1_ref, out_ref):
        out_ref[0] = (agg_ref[0] + agg_ref[1]) @ Wl1_ref[...]
        out_ref[1] = x_ref[...] @ Wr1_ref[...] + bl1_ref[...]

    full = lambda shape: pl.BlockSpec(shape, lambda i: (0,) * len(shape))
    return pl.pallas_call(
        body,
        grid=(grid,),
        in_specs=[
            pl.BlockSpec((BR, F), lambda i: (i, 0)),
            pl.BlockSpec((2, BR, F), lambda i: (0, i, 0)),
            full((F, F)), full((1, F)), full((F, F)),
        ],
        out_specs=pl.BlockSpec((2, BR, F), lambda i: (0, i, 0)),
        out_shape=jax.ShapeDtypeStruct((2, N_NODES, F), jnp.float32),
    )(x, agg_p, Wl1, bl1.reshape(1, F), Wr1)


def _tc_final(P, inv_col, w_col, Wl2, bl2, Wr2, HW1, Hb1, HW2, Hb2,
              HW3, Hb3):
    BR = 1000
    grid = N_NODES // BR

    def body(P_ref, inv_ref, w_ref,
             Wl2_ref, bl2_ref, Wr2_ref,
             HW1_ref, Hb1_ref, HW2_ref, Hb2_ref, HW3_ref, Hb3_ref,
             out_ref, acc0, acc1):
        i = pl.program_id(0)

        @pl.when(i == 0)
        def _():
            acc0[...] = jnp.zeros_like(acc0)
            acc1[...] = jnp.zeros_like(acc1)

        h = P_ref[0] * inv_ref[...] + P_ref[1]
        h = jnp.maximum(h, 0.0)
        w = w_ref[0] + w_ref[1]
        acc0[...] += jnp.sum(h.reshape(BR // 8, 8, F), axis=0)
        acc1[...] += jnp.sum((w * h).reshape(BR // 8, 8, F), axis=0)

        @pl.when(i == grid - 1)
        def _():
            s0 = jnp.sum(acc0[...], axis=0, keepdims=True) / N_NODES
            s1 = jnp.sum(acc1[...], axis=0, keepdims=True) / N_NODES
            g = s1 @ Wl2_ref[...] + bl2_ref[...] + s0 @ Wr2_ref[...]
            t = jnp.maximum(g @ HW1_ref[...] + Hb1_ref[...], 0.0)
            t = jnp.maximum(t @ HW2_ref[...] + Hb2_ref[...], 0.0)
            t = t @ HW3_ref[...] + Hb3_ref[...]
            pi = jnp.float32(math.pi)
            out_ref[...] = jnp.mod(t + pi, 2.0 * pi) - pi

    full = lambda shape: pl.BlockSpec(shape, lambda i: (0,) * len(shape))
    return pl.pallas_call(
        body,
        grid=(grid,),
        in_specs=[
            pl.BlockSpec((2, BR, F), lambda i: (0, i, 0)),
            pl.BlockSpec((BR, 1), lambda i: (i, 0)),
            pl.BlockSpec((2, BR, 1), lambda i: (0, i, 0)),
            full((F, F)), full((1, F)), full((F, F)),
            full((F, 2 * F)), full((1, 2 * F)),
            full((2 * F, 2 * F)), full((1, 2 * F)),
            full((2 * F, 8)), full((1, 8)),
        ],
        out_specs=pl.BlockSpec((1, 8), lambda i: (0, 0)),
        out_shape=jax.ShapeDtypeStruct((1, 8), jnp.float32),
        scratch_shapes=[
            pltpu.VMEM((8, F), jnp.float32),
            pltpu.VMEM((8, F), jnp.float32),
        ],
    )(P, inv_col, w_col, Wl2, bl2.reshape(1, F), Wr2,
      HW1, Hb1.reshape(1, 2 * F), HW2, Hb2.reshape(1, 2 * F),
      HW3, Hb3.reshape(1, 8))


def kernel(x, edge_index, Wl1, bl1, Wr1, Wl2, bl2, Wr2,
           HW1, Hb1, HW2, Hb2, HW3, Hb3):
    ei32 = edge_index.astype(jnp.int32)
    ei3 = ei32.reshape(2 * N_EDGES // 80, 1, 80)
    ei_flat = ei32.reshape(2 * N_EDGES)

    agg_p, cnt_p = _sc_pass1(x, ei3)
    w_p, inv = _sc_pass2(cnt_p, ei_flat)
    P = _tc_dense(x, agg_p, Wl1, bl1, Wr1)
    t = _tc_final(P, inv.reshape(N_PAD, 1), w_p.reshape(2, N_PAD, 1),
                  Wl2, bl2, Wr2, HW1, Hb1, HW2, Hb2, HW3, Hb3)
    return t[0]


# speedup vs baseline: 19.9209x; 1.0024x over previous
import functools
import math

import jax
import jax.numpy as jnp
from jax import lax
from jax.experimental import pallas as pl
from jax.experimental.pallas import tpu as pltpu
from jax.experimental.pallas import tpu_sc as plsc

N_NODES = 10000
N_EDGES = 320000
F = 128
N_PAD = 10240


def _sc_pass1(x, ei3):
    mesh = plsc.VectorSubcoreMesh(core_axis_name="c", subcore_axis_name="s")
    nc, ns = mesh.num_cores, mesh.num_subcores
    nw = nc * ns
    epw = N_EDGES // nw
    K = 80
    steps = epw // K
    assert (steps - 1) % 4 == 0
    rpt = (N_NODES // ns) // 8 * 8
    rem = N_NODES - rpt * ns
    nblk = N_EDGES // K

    RB = 128
    SW = RB * ns

    @functools.partial(
        pl.kernel,
        out_type=(
            jax.ShapeDtypeStruct((nc, N_NODES, F), jnp.float32),
            jax.ShapeDtypeStruct((nc, N_PAD), jnp.float32),
        ),
        mesh=mesh,
        scratch_types=[
            pltpu.VMEM_SHARED((N_NODES, F), jnp.float32),
            pltpu.VMEM_SHARED((ns, SW), jnp.float32),
            [pltpu.VMEM((1, K), jnp.int32) for _ in range(4)],
            [pltpu.VMEM((1, K), jnp.int32) for _ in range(4)],
            pltpu.VMEM((K, F), jnp.float32),
            pltpu.VMEM((K, F), jnp.float32),
            pltpu.VMEM((N_PAD,), jnp.float32),
            pltpu.VMEM((ns, RB), jnp.float32),
            pltpu.VMEM((RB,), jnp.float32),
            [pltpu.SemaphoreType.DMA for _ in range(4)],
            pltpu.SemaphoreType.DMA,
            pltpu.SemaphoreType.DMA,
            pltpu.SemaphoreType.DMA,
            pltpu.SemaphoreType.DMA,
        ],
        compiler_params=pltpu.CompilerParams(needs_layout_passes=False),
    )
    def k(x_hbm, ei_hbm, agg_out, cnt_out,
          agg_sh, cnt_stage, src_b, dst_b, rows0, rows1, cnt_v,
          red_buf, red_out, isem, g0, g1, ss0, ss1):
        c = lax.axis_index("c")
        s = lax.axis_index("s")
        wid = c * ns + s

        zero16 = jnp.zeros((16,), jnp.float32)
        ones16 = jnp.ones((16,), jnp.float32)

        def z0(t, _):
            rows0[t // 8, pl.ds((t % 8) * 16, 16)] = zero16
            return 0

        lax.fori_loop(0, K * F // 16, z0, 0)

        def zc(t, _):
            cnt_v[pl.ds(t * 16, 16)] = zero16
            return 0

        lax.fori_loop(0, N_PAD // 16, zc, 0)

        nfull = rpt // K
        for q in range(nfull):
            pltpu.sync_copy(rows0, agg_sh.at[pl.ds(s * rpt + q * K, K)])
        tailrows = rpt - nfull * K
        pltpu.sync_copy(rows0.at[pl.ds(0, tailrows)],
                        agg_sh.at[pl.ds(s * rpt + nfull * K, tailrows)])

        @pl.when(s == 0)
        def _():
            pltpu.sync_copy(rows0.at[pl.ds(0, rem)],
                            agg_sh.at[pl.ds(rpt * ns, rem)])

        plsc.subcore_barrier()

        rows = (rows0, rows1)
        gsem = (g0, g1)
        ssem = (ss0, ss1)
        base_blk = wid * steps

        def idx_load(i, pos):
            gb = base_blk + i
            pltpu.async_copy(ei_hbm.at[gb], src_b[pos], isem[pos])
            pltpu.async_copy(ei_hbm.at[nblk + gb], dst_b[pos], isem[pos])

        def idx_wait(pos):
            pltpu.make_async_copy(ei_hbm.at[0], src_b[pos], isem[pos]).wait()
            pltpu.make_async_copy(ei_hbm.at[0], dst_b[pos], isem[pos]).wait()

        def gather_issue(pos, rp):
            pltpu.async_copy(x_hbm.at[src_b[pos].at[0]], rows[rp], gsem[rp])

        def gather_wait(rp):
            pltpu.make_async_copy(x_hbm.at[src_b[0].at[0]], rows[rp],
                                  gsem[rp]).wait()

        def scatter_issue(pos, rp):
            pltpu.async_copy(rows[rp], agg_sh.at[dst_b[pos].at[0]], ssem[rp],
                             add=True)

        def scatter_wait(rp):
            pltpu.make_async_copy(rows[rp], agg_sh.at[dst_b[0].at[0]],
                                  ssem[rp]).wait()

        def cnt_upd(pos):
            for j in range(K // 16):
                idx = dst_b[pos][0, pl.ds(j * 16, 16)]
                plsc.addupdate_scatter(cnt_v, [idx], ones16)

        for pos in range(4):
            idx_load(pos, pos)
        idx_wait(0)
        gather_issue(0, 0)

        idx_wait(1)
        gather_issue(1, 1)
        gather_wait(0)
        scatter_issue(0, 0)
        cnt_upd(0)

        nbody = (steps - 1) // 4

        def body(p, _):
            for q in range(4):
                i = 4 * p + 1 + q
                pos = (q + 1) % 4
                nxt = (pos + 1) % 4
                rp = (q + 1) % 2
                nrp = q % 2

                @pl.when(i + 1 <= steps - 1)
                def _(pos=pos, nxt=nxt, nrp=nrp):
                    idx_wait(nxt)

                scatter_wait(nrp)

                @pl.when(i + 1 <= steps - 1)
                def _(pos=pos, nxt=nxt, nrp=nrp):
                    gather_issue(nxt, nrp)

                gather_wait(rp)
                scatter_issue(pos, rp)
                cnt_upd(pos)

                @pl.when(i + 3 <= steps - 1)
                def _(i=i, q=q):
                    idx_load(i + 3, q)

            return 0

        lax.fori_loop(0, nbody, body, 0)

        scatter_wait(0)

        plsc.subcore_barrier()
        for h in range(N_PAD // SW):
            pltpu.sync_copy(cnt_v.at[pl.ds(h * SW, SW)], cnt_stage.at[s])
            plsc.subcore_barrier()
            pltpu.sync_copy(cnt_stage.at[:, pl.ds(s * RB, RB)], red_buf)

            def red(j, _):
                acc = red_buf[0, pl.ds(j * 16, 16)]
                for r in range(1, ns):
                    acc = acc + red_buf[r, pl.ds(j * 16, 16)]
                red_out[pl.ds(j * 16, 16)] = acc
                return 0

            lax.fori_loop(0, RB // 16, red, 0)
            pltpu.sync_copy(red_out,
                            cnt_out.at[c, pl.ds(h * SW + s * RB, RB)])
            plsc.subcore_barrier()

        pltpu.sync_copy(agg_sh.at[pl.ds(s * rpt, rpt)],
                        agg_out.at[c, pl.ds(s * rpt, rpt)])

        @pl.when(s == 0)
        def _():
            pltpu.sync_copy(agg_sh.at[pl.ds(rpt * ns, rem)],
                            agg_out.at[c, pl.ds(rpt * ns, rem)])

    return k(x, ei3)


def _sc_pass2(cnt_p, ei):
    mesh = plsc.VectorSubcoreMesh(core_axis_name="c", subcore_axis_name="s")
    nc, ns = mesh.num_cores, mesh.num_subcores
    nw = nc * ns
    epw = N_EDGES // nw
    cpt = N_PAD // ns

    @functools.partial(
        pl.kernel,
        out_type=(
            jax.ShapeDtypeStruct((nc, N_PAD), jnp.float32),
            jax.ShapeDtypeStruct((N_PAD,), jnp.float32),
        ),
        mesh=mesh,
        scratch_types=[
            pltpu.VMEM_SHARED((ns, N_PAD), jnp.float32),
            pltpu.VMEM((nc, N_PAD), jnp.float32),
            pltpu.VMEM((N_PAD,), jnp.float32),
            pltpu.VMEM((N_PAD,), jnp.float32),
            pltpu.VMEM((epw,), jnp.int32),
            pltpu.VMEM((epw,), jnp.int32),
            pltpu.VMEM((ns, cpt), jnp.float32),
            pltpu.VMEM((cpt,), jnp.float32),
            pltpu.SemaphoreType.DMA,
            pltpu.SemaphoreType.DMA,
        ],
        compiler_params=pltpu.CompilerParams(needs_layout_passes=False),
    )
    def k(cnt_hbm, ei_hbm, w_out, inv_out,
          w_stage, cbuf, inv_v, w_v, src_v, dst_v, red_buf, red_out,
          csem, esem):
        c = lax.axis_index("c")
        s = lax.axis_index("s")
        wid = c * ns + s

        pltpu.async_copy(cnt_hbm, cbuf, csem)
        pltpu.async_copy(ei_hbm.at[pl.ds(wid * epw, epw)], src_v, esem)
        pltpu.async_copy(ei_hbm.at[pl.ds(N_EDGES + wid * epw, epw)], dst_v,
                         esem)

        zero16 = jnp.zeros((16,), jnp.float32)

        def zw(t, _):
            for u in range(5):
                w_v[pl.ds((t * 5 + u) * 16, 16)] = zero16
            return 0

        lax.fori_loop(0, N_PAD // 80, zw, 0)

        pltpu.make_async_copy(cnt_hbm, cbuf, csem).wait()

        def inv_body(j, _):
            for u in range(5):
                sl = pl.ds((j * 5 + u) * 16, 16)
                tot = cbuf[0, sl] + cbuf[1, sl]
                inv_v[sl] = 1.0 / jnp.maximum(tot, 1.0)
            return 0

        lax.fori_loop(0, N_PAD // 80, inv_body, 0)

        pltpu.make_async_copy(ei_hbm.at[pl.ds(0, epw)], src_v, esem).wait()
        pltpu.make_async_copy(ei_hbm.at[pl.ds(0, epw)], dst_v, esem).wait()

        def w_body(j, _):
            for u in range(5):
                sl = pl.ds((j * 5 + u) * 16, 16)
                vals = plsc.load_gather(inv_v, [dst_v[sl]])
                plsc.addupdate_scatter(w_v, [src_v[sl]], vals)
            return 0

        lax.fori_loop(0, epw // 80, w_body, 0)

        pltpu.sync_copy(w_v, w_stage.at[s])
        plsc.subcore_barrier()

        pltpu.sync_copy(w_stage.at[:, pl.ds(s * cpt, cpt)], red_buf)

        def red(j, _):
            acc = red_buf[0, pl.ds(j * 16, 16)]
            for r in range(1, ns):
                acc = acc + red_buf[r, pl.ds(j * 16, 16)]
            red_out[pl.ds(j * 16, 16)] = acc
            return 0

        lax.fori_loop(0, cpt // 16, red, 0)
        pltpu.sync_copy(red_out, w_out.at[c, pl.ds(s * cpt, cpt)])

        @pl.when(c == 0)
        def _():
            pltpu.sync_copy(inv_v.at[pl.ds(s * cpt, cpt)],
                            inv_out.at[pl.ds(s * cpt, cpt)])

    return k(cnt_p, ei)


def _tc_dense(x, agg_p, Wl1, bl1, Wr1):
    BR = 1000
    grid = N_NODES // BR

    def body(x_ref, agg_ref, Wl1_ref, bl1_ref, Wr1_ref, out_ref):
        out_ref[0] = (agg_ref[0] + agg_ref[1]) @ Wl1_ref[...]
        out_ref[1] = x_ref[...] @ Wr1_ref[...] + bl1_ref[...]

    full = lambda shape: pl.BlockSpec(shape, lambda i: (0,) * len(shape))
    return pl.pallas_call(
        body,
        grid=(grid,),
        in_specs=[
            pl.BlockSpec((BR, F), lambda i: (i, 0)),
            pl.BlockSpec((2, BR, F), lambda i: (0, i, 0)),
            full((F, F)), full((1, F)), full((F, F)),
        ],
        out_specs=pl.BlockSpec((2, BR, F), lambda i: (0, i, 0)),
        out_shape=jax.ShapeDtypeStruct((2, N_NODES, F), jnp.float32),
    )(x, agg_p, Wl1, bl1.reshape(1, F), Wr1)


def _tc_final(P, inv_col, w_col, Wl2, bl2, Wr2, HW1, Hb1, HW2, Hb2,
              HW3, Hb3):
    BR = 1000
    grid = N_NODES // BR

    def body(P_ref, inv_ref, w_ref,
             Wl2_ref, bl2_ref, Wr2_ref,
             HW1_ref, Hb1_ref, HW2_ref, Hb2_ref, HW3_ref, Hb3_ref,
             out_ref, acc0, acc1):
        i = pl.program_id(0)

        @pl.when(i == 0)
        def _():
            acc0[...] = jnp.zeros_like(acc0)
            acc1[...] = jnp.zeros_like(acc1)

        h = P_ref[0] * inv_ref[...] + P_ref[1]
        h = jnp.maximum(h, 0.0)
        w = w_ref[0] + w_ref[1]
        acc0[...] += jnp.sum(h.reshape(BR // 8, 8, F), axis=0)
        acc1[...] += jnp.sum((w * h).reshape(BR // 8, 8, F), axis=0)

        @pl.when(i == grid - 1)
        def _():
            s0 = jnp.sum(acc0[...], axis=0, keepdims=True) / N_NODES
            s1 = jnp.sum(acc1[...], axis=0, keepdims=True) / N_NODES
            g = s1 @ Wl2_ref[...] + bl2_ref[...] + s0 @ Wr2_ref[...]
            t = jnp.maximum(g @ HW1_ref[...] + Hb1_ref[...], 0.0)
            t = jnp.maximum(t @ HW2_ref[...] + Hb2_ref[...], 0.0)
            t = t @ HW3_ref[...] + Hb3_ref[...]
            pi = jnp.float32(math.pi)
            out_ref[...] = jnp.mod(t + pi, 2.0 * pi) - pi

    full = lambda shape: pl.BlockSpec(shape, lambda i: (0,) * len(shape))
    return pl.pallas_call(
        body,
        grid=(grid,),
        in_specs=[
            pl.BlockSpec((2, BR, F), lambda i: (0, i, 0)),
            pl.BlockSpec((BR, 1), lambda i: (i, 0)),
            pl.BlockSpec((2, BR, 1), lambda i: (0, i, 0)),
            full((F, F)), full((1, F)), full((F, F)),
            full((F, 2 * F)), full((1, 2 * F)),
            full((2 * F, 2 * F)), full((1, 2 * F)),
            full((2 * F, 8)), full((1, 8)),
        ],
        out_specs=pl.BlockSpec((1, 8), lambda i: (0, 0)),
        out_shape=jax.ShapeDtypeStruct((1, 8), jnp.float32),
        scratch_shapes=[
            pltpu.VMEM((8, F), jnp.float32),
            pltpu.VMEM((8, F), jnp.float32),
        ],
    )(P, inv_col, w_col, Wl2, bl2.reshape(1, F), Wr2,
      HW1, Hb1.reshape(1, 2 * F), HW2, Hb2.reshape(1, 2 * F),
      HW3, Hb3.reshape(1, 8))


def kernel(x, edge_index, Wl1, bl1, Wr1, Wl2, bl2, Wr2,
           HW1, Hb1, HW2, Hb2, HW3, Hb3):
    ei32 = edge_index.astype(jnp.int32)
    ei3 = ei32.reshape(2 * N_EDGES // 80, 1, 80)
    ei_flat = ei32.reshape(2 * N_EDGES)

    agg_p, cnt_p = _sc_pass1(x, ei3)
    P = _tc_dense(x, agg_p, Wl1, bl1, Wr1)
    w_p, inv = _sc_pass2(cnt_p, ei_flat)
    t = _tc_final(P, inv.reshape(N_PAD, 1), w_p.reshape(2, N_PAD, 1),
                  Wl2, bl2, Wr2, HW1, Hb1, HW2, Hb2, HW3, Hb3)
    return t[0]
